# baseline (device time: 22206 ns/iter reference)
import jax
import jax.numpy as jnp
from jax import lax
from jax.experimental import pallas as pl
from jax.experimental.pallas import tpu as pltpu

N_DEV = 4


def kernel(x, Wg, Wu, Wd):
    m, _ = x.shape
    n = Wd.shape[1]

    def body(x_ref, wg_ref, wu_ref, wd_ref, out_ref, comm_ref, send_sems, recv_sems):
        my = lax.axis_index("i")
        left = lax.rem(my + N_DEV - 1, N_DEV)
        right = lax.rem(my + 1, N_DEV)

        barrier_sem = pltpu.get_barrier_semaphore()
        for nbr in (left, right):
            pl.semaphore_signal(
                barrier_sem, inc=1,
                device_id=(nbr,), device_id_type=pl.DeviceIdType.MESH,
            )
        pl.semaphore_wait(barrier_sem, 2)

        xv = x_ref[:, :]
        gate = jnp.dot(xv, wg_ref[:, :], preferred_element_type=jnp.float32)
        up = jnp.dot(xv, wu_ref[:, :], preferred_element_type=jnp.float32)
        hidden = gate * (up * jax.nn.sigmoid(up))
        partial = jnp.dot(hidden, wd_ref[:, :], preferred_element_type=jnp.float32)

        comm_ref[0, :, :] = partial
        out_ref[:, :] = partial

        for hop in range(N_DEV - 1):
            rdma = pltpu.make_async_remote_copy(
                src_ref=comm_ref.at[hop],
                dst_ref=comm_ref.at[hop + 1],
                send_sem=send_sems.at[hop],
                recv_sem=recv_sems.at[hop],
                device_id=(right,),
                device_id_type=pl.DeviceIdType.MESH,
            )
            rdma.start()
            rdma.wait()
            out_ref[:, :] = out_ref[:, :] + comm_ref[hop + 1, :, :]

    return pl.pallas_call(
        body,
        out_shape=jax.ShapeDtypeStruct((m, n), jnp.float32),
        in_specs=[pl.BlockSpec(memory_space=pltpu.VMEM)] * 4,
        out_specs=pl.BlockSpec(memory_space=pltpu.VMEM),
        scratch_shapes=[
            pltpu.VMEM((N_DEV, m, n), jnp.float32),
            pltpu.SemaphoreType.DMA((N_DEV - 1,)),
            pltpu.SemaphoreType.DMA((N_DEV - 1,)),
        ],
        compiler_params=pltpu.CompilerParams(collective_id=0),
    )(x, Wg, Wu, Wd)


# device time: 16108 ns/iter; 1.3786x vs baseline; 1.3786x over previous
import jax
import jax.numpy as jnp
from jax import lax
from jax.experimental import pallas as pl
from jax.experimental.pallas import tpu as pltpu

N_DEV = 4
OFFSETS = (1, -1, 2)


def kernel(x, Wg, Wu, Wd):
    m, _ = x.shape
    n = Wd.shape[1]

    def body(x_ref, wg_ref, wu_ref, wd_ref, out_ref, src_ref, recv_ref,
             send_sems, recv_sems):
        my = lax.axis_index("i")
        peers = [lax.rem(my + d + N_DEV, N_DEV) for d in OFFSETS]

        barrier_sem = pltpu.get_barrier_semaphore()
        for p in peers:
            pl.semaphore_signal(
                barrier_sem, inc=1,
                device_id=(p,), device_id_type=pl.DeviceIdType.MESH,
            )
        pl.semaphore_wait(barrier_sem, len(peers))

        xv = x_ref[:, :]
        gate = jnp.dot(xv, wg_ref[:, :], preferred_element_type=jnp.float32)
        up = jnp.dot(xv, wu_ref[:, :], preferred_element_type=jnp.float32)
        hidden = gate * (up * jax.nn.sigmoid(up))
        partial = jnp.dot(hidden, wd_ref[:, :], preferred_element_type=jnp.float32)

        src_ref[:, :] = partial

        rdmas = []
        for j, p in enumerate(peers):
            rdma = pltpu.make_async_remote_copy(
                src_ref=src_ref,
                dst_ref=recv_ref.at[j],
                send_sem=send_sems.at[j],
                recv_sem=recv_sems.at[j],
                device_id=(p,),
                device_id_type=pl.DeviceIdType.MESH,
            )
            rdma.start()
            rdmas.append(rdma)

        out_ref[:, :] = partial
        for j, rdma in enumerate(rdmas):
            rdma.wait_recv()
            out_ref[:, :] = out_ref[:, :] + recv_ref[j, :, :]
        for rdma in rdmas:
            rdma.wait_send()

    return pl.pallas_call(
        body,
        out_shape=jax.ShapeDtypeStruct((m, n), jnp.float32),
        in_specs=[pl.BlockSpec(memory_space=pltpu.VMEM)] * 4,
        out_specs=pl.BlockSpec(memory_space=pltpu.VMEM),
        scratch_shapes=[
            pltpu.VMEM((m, n), jnp.float32),
            pltpu.VMEM((3, m, n), jnp.float32),
            pltpu.SemaphoreType.DMA((3,)),
            pltpu.SemaphoreType.DMA((3,)),
        ],
        compiler_params=pltpu.CompilerParams(collective_id=0),
    )(x, Wg, Wu, Wd)
